# baseline (speedup 1.0000x reference)
import functools

import jax
import jax.numpy as jnp
from jax import lax
from jax.experimental import pallas as pl
from jax.experimental.pallas import tpu as pltpu
from jax.experimental.pallas import tpu_sc as plsc

_HI = jax.lax.Precision.HIGHEST


def _top3(q, c, T, S):
    qn = jnp.sum(q * q, axis=1)
    cn = jnp.sum(c * c, axis=1)
    cross = jax.lax.dot_general(
        q.astype(jnp.bfloat16), c.astype(jnp.bfloat16),
        (((1,), (1,)), ((), ())),
        preferred_element_type=jnp.float32)
    d = (qn[:, None] - 2.0 * cross) + cn[None, :]
    inf = jnp.float32(jnp.inf)
    iota = jax.lax.broadcasted_iota(jnp.int32, (T, S), 1)
    m1 = jnp.min(d, axis=1)
    eq1 = d == m1[:, None]
    i1 = jnp.min(jnp.where(eq1, iota, S), axis=1)
    d1 = jnp.where(eq1, inf, d)
    m2 = jnp.min(d1, axis=1)
    eq2 = d1 == m2[:, None]
    i2 = jnp.min(jnp.where(eq2, iota, S), axis=1)
    d2 = jnp.where(eq2, inf, d1)
    m3 = jnp.min(d2, axis=1)
    eq3 = d2 == m3[:, None]
    i3 = jnp.min(jnp.where(eq3, iota, S), axis=1)
    r1 = 1.0 / (m1 + 1e-8)
    r2 = 1.0 / (m2 + 1e-8)
    r3 = 1.0 / (m3 + 1e-8)
    norm = (r1 + r2) + r3
    return (i1, i2, i3), (r1 / norm, r2 / norm, r3 / norm)


def _stage_a1(xyz1_ref, xyz2_ref, idx_ref, w_ref, *, T, S, B):
    b = pl.program_id(0)
    q = xyz1_ref[0]
    idx_rows = []
    w_rows = []
    for l in (1, 0):
        iss, ws = _top3(q, xyz2_ref[l, 0], T, S)
        base = (l * B + b) * S
        idx_rows.extend([ii + base for ii in iss])
        w_rows.extend(list(ws))
    zi = jnp.zeros_like(idx_rows[0])
    zw = jnp.zeros_like(w_rows[0])
    idx_ref[...] = jnp.stack(idx_rows + [zi, zi], axis=0)
    w_ref[...] = jnp.stack(w_rows + [zw, zw], axis=0)


def _make_sc_gather(BN, D):
    info = plsc.get_sparse_core_info()
    NC, NS = info.num_cores, info.num_subcores
    NW = NC * NS
    per_w = BN // NW
    C = 64
    n_chunks = per_w // C
    mesh = plsc.VectorSubcoreMesh(core_axis_name="c", subcore_axis_name="s")

    @functools.partial(
        pl.kernel, mesh=mesh,
        out_type=jax.ShapeDtypeStruct((BN, 2 * D), jnp.float32),
        scratch_types=[
            pltpu.VMEM((6, C), jnp.int32),
            pltpu.VMEM((6, C, 128), jnp.float32),
            pltpu.VMEM((6, C), jnp.float32),
            pltpu.VMEM((C, 2 * D), jnp.float32),
            pltpu.SemaphoreType.DMA,
        ],
    )
    def gather_k(table_hbm, idx_hbm, w_hbm, out_hbm,
                 idx_v, rows_v, w_v, out_v, sem):
        wid = lax.axis_index("s") * NC + lax.axis_index("c")
        base = wid * per_w

        def chunk(ci, carry):
            c0 = base + ci * C
            for k in range(6):
                pltpu.sync_copy(idx_hbm.at[k, pl.ds(c0, C)], idx_v.at[k])
                pltpu.sync_copy(w_hbm.at[k, pl.ds(c0, C)], w_v.at[k])
            copies = [pltpu.async_copy(table_hbm.at[idx_v.at[k]],
                                       rows_v.at[k], sem)
                      for k in range(6)]
            for cp in copies:
                cp.wait()
            dnums = lax.GatherDimensionNumbers(
                offset_dims=(), collapsed_slice_dims=(0,),
                start_index_map=(0,))
            for g in range(C // 16):
                wv = [w_v[k, pl.ds(g * 16, 16)] for k in range(6)]
                for i in range(16):
                    p = g * 16 + i
                    pv = jnp.full((16, 1), i, jnp.int32)
                    s = [lax.gather(
                        wv[k], pv, dnums, (1,),
                        mode=lax.GatherScatterMode.PROMISE_IN_BOUNDS)
                        for k in range(6)]
                    for seg in range(D // 16):
                        sl = pl.ds(seg * 16, 16)
                        r = [rows_v[k, p, sl] for k in range(6)]
                        lvl1 = (r[0] * s[0] + r[1] * s[1]) + r[2] * s[2]
                        lvl0 = (r[3] * s[3] + r[4] * s[4]) + r[5] * s[5]
                        out_v[p, sl] = lvl1
                        out_v[p, pl.ds(D + seg * 16, 16)] = lvl0
            pltpu.sync_copy(out_v, out_hbm.at[pl.ds(c0, C)])
            return carry

        lax.fori_loop(0, n_chunks, chunk, 0)

    return gather_k


def _stage_a2(p1_ref, it_ref, w0_ref, b0_ref, x1_ref, st_ref):
    w0 = w0_ref[...]
    D1 = p1_ref.shape[1]
    x1 = jax.lax.dot_general(
        w0[:, :D1].astype(jnp.bfloat16), p1_ref[0].astype(jnp.bfloat16),
        (((1,), (0,)), ((), ())), preferred_element_type=jnp.float32)
    x1 = x1 + jax.lax.dot_general(
        w0[:, D1:].astype(jnp.bfloat16), it_ref[0].astype(jnp.bfloat16),
        (((1,), (1,)), ((), ())), preferred_element_type=jnp.float32)
    x1 = x1 + b0_ref[...]
    x1_ref[0] = x1
    first = (pl.program_id(0) == 0) & (pl.program_id(1) == 0)

    @pl.when(first)
    def _():
        st_ref[...] = jnp.zeros_like(st_ref)

    s = jnp.sum(x1, axis=1, keepdims=True)
    sq = jnp.sum(x1 * x1, axis=1, keepdims=True)
    st_ref[...] += jnp.concatenate([s, sq], axis=1)


def _stage_b(x1_ref, st1_ref, g0_ref, bt0_ref, w1_ref, b1_ref,
             x2_ref, st_ref, *, count):
    st = st1_ref[...]
    mean = st[:, 0:1] * (1.0 / count)
    var = st[:, 1:2] * (1.0 / count) - mean * mean
    rstd = 1.0 / jnp.sqrt(var + 1e-5)
    h = (x1_ref[0] - mean) * (rstd * g0_ref[...]) + bt0_ref[...]
    h = jnp.maximum(h, 0.0)
    x2 = jax.lax.dot_general(
        w1_ref[...].astype(jnp.bfloat16), h.astype(jnp.bfloat16),
        (((1,), (0,)), ((), ())), preferred_element_type=jnp.float32)
    x2 = x2 + b1_ref[...]
    x2_ref[0] = x2
    first = (pl.program_id(0) == 0) & (pl.program_id(1) == 0)

    @pl.when(first)
    def _():
        st_ref[...] = jnp.zeros_like(st_ref)

    s = jnp.sum(x2, axis=1, keepdims=True)
    sq = jnp.sum(x2 * x2, axis=1, keepdims=True)
    st_ref[...] += jnp.concatenate([s, sq], axis=1)


def _stage_c(x2_ref, st2_ref, g1_ref, bt1_ref, out_ref, *, count):
    st = st2_ref[...]
    mean = st[:, 0:1] * (1.0 / count)
    var = st[:, 1:2] * (1.0 / count) - mean * mean
    rstd = 1.0 / jnp.sqrt(var + 1e-5)
    y = (x2_ref[0] - mean) * (rstd * g1_ref[...]) + bt1_ref[...]
    out_ref[0] = jnp.maximum(y, 0.0)


def kernel(xyz1, xyz2_list, points1, points2_list,
           conv_w0, conv_b0, gamma0, beta0,
           conv_w1, conv_b1, gamma1, beta1):
    B, _, N = xyz1.shape
    L, _, _, S = xyz2_list.shape
    D1 = points1.shape[1]
    D2 = points2_list.shape[2]
    C1 = conv_w0.shape[0]
    C2 = conv_w1.shape[0]
    T = 512 if N % 512 == 0 else N
    NT = N // T
    BN = B * N
    count = float(BN)

    col = lambda v: v.reshape(-1, 1)
    b0, g0, bt0 = col(conv_b0), col(gamma0), col(beta0)
    b1, g1, bt1 = col(conv_b1), col(gamma1), col(beta1)
    xyz1_t = jnp.transpose(xyz1, (0, 2, 1))
    xyz2_t = jnp.transpose(xyz2_list, (0, 1, 3, 2))
    table = jnp.transpose(points2_list, (0, 1, 3, 2)).reshape(L * B * S, D2)
    table = jnp.pad(table, ((0, 0), (0, 128 - D2)))

    grid = (B, NT)
    full2 = lambda b, n: (0, 0)

    idx_all, w_all = pl.pallas_call(
        functools.partial(_stage_a1, T=T, S=S, B=B),
        grid=grid,
        in_specs=[
            pl.BlockSpec((1, T, 3), lambda b, n: (b, n, 0)),
            pl.BlockSpec((L, 1, S, 3), lambda b, n: (0, b, 0, 0)),
        ],
        out_specs=[
            pl.BlockSpec((8, T), lambda b, n: (0, b * NT + n)),
            pl.BlockSpec((8, T), lambda b, n: (0, b * NT + n)),
        ],
        out_shape=[
            jax.ShapeDtypeStruct((8, BN), jnp.int32),
            jax.ShapeDtypeStruct((8, BN), jnp.float32),
        ],
    )(xyz1_t, xyz2_t)

    interp = _make_sc_gather(BN, D2)(table, idx_all, w_all)
    interp = interp.reshape(B, N, 2 * D2)

    x1, st1 = pl.pallas_call(
        _stage_a2,
        grid=grid,
        in_specs=[
            pl.BlockSpec((1, D1, T), lambda b, n: (b, 0, n)),
            pl.BlockSpec((1, T, L * D2), lambda b, n: (b, n, 0)),
            pl.BlockSpec((C1, D1 + L * D2), full2),
            pl.BlockSpec((C1, 1), full2),
        ],
        out_specs=[
            pl.BlockSpec((1, C1, T), lambda b, n: (b, 0, n)),
            pl.BlockSpec((C1, 2), full2),
        ],
        out_shape=[
            jax.ShapeDtypeStruct((B, C1, N), jnp.float32),
            jax.ShapeDtypeStruct((C1, 2), jnp.float32),
        ],
    )(points1, interp, conv_w0, b0)

    x2, st2 = pl.pallas_call(
        functools.partial(_stage_b, count=count),
        grid=grid,
        in_specs=[
            pl.BlockSpec((1, C1, T), lambda b, n: (b, 0, n)),
            pl.BlockSpec((C1, 2), full2),
            pl.BlockSpec((C1, 1), full2),
            pl.BlockSpec((C1, 1), full2),
            pl.BlockSpec((C2, C1), full2),
            pl.BlockSpec((C2, 1), full2),
        ],
        out_specs=[
            pl.BlockSpec((1, C2, T), lambda b, n: (b, 0, n)),
            pl.BlockSpec((C2, 2), full2),
        ],
        out_shape=[
            jax.ShapeDtypeStruct((B, C2, N), jnp.float32),
            jax.ShapeDtypeStruct((C2, 2), jnp.float32),
        ],
    )(x1, st1, g0, bt0, conv_w1, b1)

    out = pl.pallas_call(
        functools.partial(_stage_c, count=count),
        grid=grid,
        in_specs=[
            pl.BlockSpec((1, C2, T), lambda b, n: (b, 0, n)),
            pl.BlockSpec((C2, 2), full2),
            pl.BlockSpec((C2, 1), full2),
            pl.BlockSpec((C2, 1), full2),
        ],
        out_specs=pl.BlockSpec((1, C2, T), lambda b, n: (b, 0, n)),
        out_shape=jax.ShapeDtypeStruct((B, C2, N), jnp.float32),
    )(x2, st2, g1, bt1)

    return out


# speedup vs baseline: 2.0636x; 2.0636x over previous
import functools

import jax
import jax.numpy as jnp
from jax.experimental import pallas as pl

_HI = jax.lax.Precision.HIGHEST


def _top3_weights(q, c, T, S):
    qn = jnp.sum(q * q, axis=1)
    cn = jnp.sum(c * c, axis=1)
    cross = jax.lax.dot_general(
        q.astype(jnp.bfloat16), c.astype(jnp.bfloat16),
        (((1,), (1,)), ((), ())),
        preferred_element_type=jnp.float32)
    d = (qn[:, None] - 2.0 * cross) + cn[None, :]
    inf = jnp.float32(jnp.inf)
    one = jnp.float32(1.0)
    zero = jnp.float32(0.0)
    bf = jnp.bfloat16
    m1 = jnp.min(d, axis=1)
    eq1 = d == m1[:, None]
    oh1 = jnp.where(eq1, one, zero).astype(bf)
    d1 = jnp.where(eq1, inf, d)
    m2 = jnp.min(d1, axis=1)
    eq2 = d1 == m2[:, None]
    oh2 = jnp.where(eq2, one, zero).astype(bf)
    d2 = jnp.where(eq2, inf, d1)
    m3 = jnp.min(d2, axis=1)
    oh3 = jnp.where(d2 == m3[:, None], one, zero).astype(bf)
    r1 = 1.0 / (m1 + 1e-8)
    r2 = 1.0 / (m2 + 1e-8)
    r3 = 1.0 / (m3 + 1e-8)
    norm = (r1 + r2) + r3
    return (oh1, oh2, oh3), (r1 / norm, r2 / norm, r3 / norm)


def _stage_a(xyz1_ref, xyz2_ref, p1_ref, p2_ref, w0_ref, b0_ref,
             x1_ref, st_ref, *, T, S):
    q = xyz1_ref[0]
    feats = [p1_ref[0]]
    for l in (1, 0):
        ohs, ws = _top3_weights(q, xyz2_ref[l, 0], T, S)
        pts = p2_ref[l, 0]
        hi = pts.astype(jnp.bfloat16)
        rem = pts - hi.astype(jnp.float32)
        mid = rem.astype(jnp.bfloat16)
        lo = (rem - mid.astype(jnp.float32)).astype(jnp.bfloat16)
        parts = jnp.concatenate([hi, mid, lo], axis=0)
        gs = []
        for oh in ohs:
            g3 = jax.lax.dot_general(
                parts, oh, (((1,), (1,)), ((), ())),
                preferred_element_type=jnp.float32)
            D = pts.shape[0]
            gs.append((g3[:D] + g3[D:2 * D]) + g3[2 * D:])
        interp_t = ((gs[0] * ws[0][None, :] + gs[1] * ws[1][None, :])
                    + gs[2] * ws[2][None, :])
        feats.append(interp_t)
    feat_t = jnp.concatenate(feats, axis=0)
    x1 = jax.lax.dot_general(
        w0_ref[...].astype(jnp.bfloat16), feat_t.astype(jnp.bfloat16),
        (((1,), (0,)), ((), ())), preferred_element_type=jnp.float32)
    x1 = x1 + b0_ref[...]
    x1_ref[0] = x1
    first = (pl.program_id(0) == 0) & (pl.program_id(1) == 0)

    @pl.when(first)
    def _():
        st_ref[...] = jnp.zeros_like(st_ref)

    s = jnp.sum(x1, axis=1, keepdims=True)
    sq = jnp.sum(x1 * x1, axis=1, keepdims=True)
    st_ref[...] += jnp.concatenate([s, sq], axis=1)


def _stage_b(x1_ref, st1_ref, g0_ref, bt0_ref, w1_ref, b1_ref,
             x2_ref, st_ref, *, count):
    st = st1_ref[...]
    mean = st[:, 0:1] * (1.0 / count)
    var = st[:, 1:2] * (1.0 / count) - mean * mean
    rstd = 1.0 / jnp.sqrt(var + 1e-5)
    h = (x1_ref[0] - mean) * (rstd * g0_ref[...]) + bt0_ref[...]
    h = jnp.maximum(h, 0.0)
    x2 = jax.lax.dot_general(
        w1_ref[...].astype(jnp.bfloat16), h.astype(jnp.bfloat16),
        (((1,), (0,)), ((), ())), preferred_element_type=jnp.float32)
    x2 = x2 + b1_ref[...]
    x2_ref[0] = x2
    first = (pl.program_id(0) == 0) & (pl.program_id(1) == 0)

    @pl.when(first)
    def _():
        st_ref[...] = jnp.zeros_like(st_ref)

    s = jnp.sum(x2, axis=1, keepdims=True)
    sq = jnp.sum(x2 * x2, axis=1, keepdims=True)
    st_ref[...] += jnp.concatenate([s, sq], axis=1)


def _stage_c(x2_ref, st2_ref, g1_ref, bt1_ref, out_ref, *, count):
    st = st2_ref[...]
    mean = st[:, 0:1] * (1.0 / count)
    var = st[:, 1:2] * (1.0 / count) - mean * mean
    rstd = 1.0 / jnp.sqrt(var + 1e-5)
    y = (x2_ref[0] - mean) * (rstd * g1_ref[...]) + bt1_ref[...]
    out_ref[0] = jnp.maximum(y, 0.0)


def kernel(xyz1, xyz2_list, points1, points2_list,
           conv_w0, conv_b0, gamma0, beta0,
           conv_w1, conv_b1, gamma1, beta1):
    B, _, N = xyz1.shape
    L, _, _, S = xyz2_list.shape
    D1 = points1.shape[1]
    D2 = points2_list.shape[2]
    C1 = conv_w0.shape[0]
    C2 = conv_w1.shape[0]
    T = 1024 if N % 1024 == 0 else N
    NT = N // T
    count = float(B * N)

    col = lambda v: v.reshape(-1, 1)
    b0, g0, bt0 = col(conv_b0), col(gamma0), col(beta0)
    b1, g1, bt1 = col(conv_b1), col(gamma1), col(beta1)
    xyz1_t = jnp.transpose(xyz1, (0, 2, 1))
    xyz2_t = jnp.transpose(xyz2_list, (0, 1, 3, 2))

    grid = (B, NT)
    full2 = lambda b, n: (0, 0)

    x1, st1 = pl.pallas_call(
        functools.partial(_stage_a, T=T, S=S),
        grid=grid,
        in_specs=[
            pl.BlockSpec((1, T, 3), lambda b, n: (b, n, 0)),
            pl.BlockSpec((L, 1, S, 3), lambda b, n: (0, b, 0, 0)),
            pl.BlockSpec((1, D1, T), lambda b, n: (b, 0, n)),
            pl.BlockSpec((L, 1, D2, S), lambda b, n: (0, b, 0, 0)),
            pl.BlockSpec((C1, D1 + L * D2), full2),
            pl.BlockSpec((C1, 1), full2),
        ],
        out_specs=[
            pl.BlockSpec((1, C1, T), lambda b, n: (b, 0, n)),
            pl.BlockSpec((C1, 2), full2),
        ],
        out_shape=[
            jax.ShapeDtypeStruct((B, C1, N), jnp.float32),
            jax.ShapeDtypeStruct((C1, 2), jnp.float32),
        ],
    )(xyz1_t, xyz2_t, points1, points2_list, conv_w0, b0)

    x2, st2 = pl.pallas_call(
        functools.partial(_stage_b, count=count),
        grid=grid,
        in_specs=[
            pl.BlockSpec((1, C1, T), lambda b, n: (b, 0, n)),
            pl.BlockSpec((C1, 2), full2),
            pl.BlockSpec((C1, 1), full2),
            pl.BlockSpec((C1, 1), full2),
            pl.BlockSpec((C2, C1), full2),
            pl.BlockSpec((C2, 1), full2),
        ],
        out_specs=[
            pl.BlockSpec((1, C2, T), lambda b, n: (b, 0, n)),
            pl.BlockSpec((C2, 2), full2),
        ],
        out_shape=[
            jax.ShapeDtypeStruct((B, C2, N), jnp.float32),
            jax.ShapeDtypeStruct((C2, 2), jnp.float32),
        ],
    )(x1, st1, g0, bt0, conv_w1, b1)

    out = pl.pallas_call(
        functools.partial(_stage_c, count=count),
        grid=grid,
        in_specs=[
            pl.BlockSpec((1, C2, T), lambda b, n: (b, 0, n)),
            pl.BlockSpec((C2, 2), full2),
            pl.BlockSpec((C2, 1), full2),
            pl.BlockSpec((C2, 1), full2),
        ],
        out_specs=pl.BlockSpec((1, C2, T), lambda b, n: (b, 0, n)),
        out_shape=jax.ShapeDtypeStruct((B, C2, N), jnp.float32),
    )(x2, st2, g1, bt1)

    return out
